# baseline (speedup 1.0000x reference)
import functools

import jax
import jax.numpy as jnp
from jax import lax
from jax.experimental import pallas as pl
from jax.experimental.pallas import tpu as pltpu
from jax.experimental.pallas import tpu_sc as plsc

N = 10000
E = 320000
D_IN = 128
D_HID = 16
D_OUT = 8

NC = 2
NS = 16
NW = NC * NS
LANE = 128
RPW = 80
EPAD = NW * RPW * LANE
NPAD = 10112
RPT = NPAD // NS
NDEG = NPAD
DPT = NDEG // NS
GRP = 8
NGRP = RPW // GRP

_mesh = plsc.VectorSubcoreMesh(core_axis_name="c", subcore_axis_name="s")
_params = pltpu.CompilerParams(use_tc_tiling_on_sc=False)


@functools.partial(
    pl.kernel,
    out_type=jax.ShapeDtypeStruct((NC * NDEG,), jnp.float32),
    mesh=_mesh,
    compiler_params=_params,
    scratch_types=[
        pltpu.VMEM((RPW, LANE), jnp.int32),
        pltpu.VMEM((LANE,), jnp.float32),
        pltpu.VMEM((DPT,), jnp.float32),
        pltpu.VMEM_SHARED((NDEG,), jnp.float32),
        pltpu.SemaphoreType.DMA,
    ],
)
def _sc_degree(dst_hbm, zeros_hbm, ones_hbm, out_hbm, idx_v, ones_v, buf_v, acc_s, sem):
    c = lax.axis_index("c")
    s = lax.axis_index("s")
    wid = c * NS + s
    pltpu.sync_copy(ones_hbm, ones_v)
    pltpu.sync_copy(dst_hbm.at[pl.ds(wid * RPW, RPW), :], idx_v)
    pltpu.sync_copy(zeros_hbm.at[pl.ds(s * DPT, DPT)], buf_v)
    pltpu.sync_copy(buf_v, acc_s.at[pl.ds(s * DPT, DPT)])
    plsc.subcore_barrier()

    @pl.loop(0, NGRP)
    def _(g):
        for b in range(GRP):
            pltpu.async_copy(ones_v, acc_s.at[idx_v.at[g * GRP + b]], sem, add=True)

    @pl.loop(0, NGRP)
    def _(g):
        for b in range(GRP):
            pltpu.make_async_copy(ones_v, acc_s.at[idx_v.at[g * GRP + b]], sem).wait()

    plsc.subcore_barrier()
    pltpu.sync_copy(acc_s.at[pl.ds(s * DPT, DPT)], buf_v)
    pltpu.sync_copy(buf_v, out_hbm.at[pl.ds(c * NDEG + s * DPT, DPT)])


def _make_sc_layer(F):
    @functools.partial(
        pl.kernel,
        out_type=jax.ShapeDtypeStruct((NC, NPAD, F), jnp.float32),
        mesh=_mesh,
        compiler_params=_params,
        scratch_types=[
            pltpu.VMEM((RPW, LANE), jnp.int32),
            pltpu.VMEM((RPW, LANE), jnp.int32),
            pltpu.VMEM((2, GRP, LANE, F), jnp.float32),
            pltpu.VMEM((RPT, F), jnp.float32),
            pltpu.VMEM_SHARED((NPAD, F), jnp.float32),
            pltpu.SemaphoreType.DMA,
            pltpu.SemaphoreType.DMA,
        ],
    )
    def _sc_layer(tbl_hbm, src_hbm, dst_hbm, zeros_hbm, out_hbm,
                  src_v, dst_v, msg_v, buf_v, acc_s, gsem, ssem):
        c = lax.axis_index("c")
        s = lax.axis_index("s")
        wid = c * NS + s
        rows = pl.ds(s * RPT, RPT)
        pltpu.sync_copy(src_hbm.at[pl.ds(wid * RPW, RPW), :], src_v)
        pltpu.sync_copy(dst_hbm.at[pl.ds(wid * RPW, RPW), :], dst_v)

        @pl.when(c == 0)
        def _():
            pltpu.sync_copy(tbl_hbm.at[rows, :], buf_v)

        @pl.when(c == 1)
        def _():
            pltpu.sync_copy(zeros_hbm.at[rows, :], buf_v)

        pltpu.sync_copy(buf_v, acc_s.at[rows, :])
        plsc.subcore_barrier()

        def fire_gathers(g, slot):
            for b in range(GRP):
                pltpu.async_copy(
                    tbl_hbm.at[src_v.at[g * GRP + b]], msg_v.at[slot, b], gsem)

        fire_gathers(0, 0)

        @pl.loop(0, NGRP)
        def _(g):
            cur = lax.rem(g, 2)
            nxt = lax.rem(g + 1, 2)

            @pl.when(g + 1 < NGRP)
            def _():
                fire_gathers(g + 1, nxt)

            for b in range(GRP):
                pltpu.make_async_copy(
                    tbl_hbm.at[src_v.at[g * GRP + b]], msg_v.at[cur, b], gsem).wait()
                pltpu.async_copy(
                    msg_v.at[cur, b], acc_s.at[dst_v.at[g * GRP + b]], ssem, add=True)
            for b in range(GRP):
                pltpu.make_async_copy(
                    msg_v.at[cur, b], acc_s.at[dst_v.at[g * GRP + b]], ssem).wait()

        plsc.subcore_barrier()
        pltpu.sync_copy(acc_s.at[rows, :], buf_v)
        pltpu.sync_copy(buf_v, out_hbm.at[c, rows, :])

    return _sc_layer


_sc_layer16 = _make_sc_layer(D_HID)
_sc_layer8 = _make_sc_layer(D_OUT)


def _tc_h1_body(x_ref, w1_ref, h1_ref):
    h = jnp.dot(x_ref[...], w1_ref[...], preferred_element_type=jnp.float32)
    h1_ref[:N, :] = h
    h1_ref[N:, :] = jnp.zeros((NPAD - N, D_HID), jnp.float32)


def _tc_scale_body(deg_ref, h1_ref, dinv_ref, h1p_ref):
    deg = deg_ref[0] + deg_ref[1] + 1.0
    dinv = lax.rsqrt(deg)
    dinv_ref[...] = dinv
    h1p_ref[...] = h1_ref[...] * dinv


def _tc_b_body(p_ref, dinv_ref, b1_ref, w2_ref, h2p_ref):
    sacc = p_ref[0] + p_ref[1]
    dinv = dinv_ref[...]
    z = jnp.maximum(sacc * dinv + b1_ref[...], 0.0)
    h2p_ref[...] = jnp.dot(z, w2_ref[...], preferred_element_type=jnp.float32) * dinv


def _tc_d_body(p_ref, dinv_ref, b2_ref, out_ref):
    sacc = p_ref[0, :N, :] + p_ref[1, :N, :]
    out_ref[...] = sacc * dinv_ref[:N] + b2_ref[...]


def kernel(x, edge_index, W1, b1, W2, b2):
    pad = EPAD - E
    src2d = jnp.concatenate(
        [edge_index[0], jnp.zeros((pad,), jnp.int32)]).reshape(NW * RPW, LANE)
    dst2d = jnp.concatenate(
        [edge_index[1], jnp.full((pad,), N, jnp.int32)]).reshape(NW * RPW, LANE)
    zeros1d = jnp.zeros((NDEG,), jnp.float32)
    ones = jnp.ones((LANE,), jnp.float32)
    zeros16 = jnp.zeros((NPAD, D_HID), jnp.float32)
    zeros8 = jnp.zeros((NPAD, D_OUT), jnp.float32)

    deg_part = _sc_degree(dst2d, zeros1d, ones)
    h1 = pl.pallas_call(
        _tc_h1_body,
        out_shape=jax.ShapeDtypeStruct((NPAD, D_HID), jnp.float32),
    )(x, W1)
    degp = deg_part.reshape(NC, NPAD, 1)

    dinv, h1p = pl.pallas_call(
        _tc_scale_body,
        out_shape=(
            jax.ShapeDtypeStruct((NPAD, 1), jnp.float32),
            jax.ShapeDtypeStruct((NPAD, D_HID), jnp.float32),
        ),
    )(degp, h1)

    p1 = _sc_layer16(h1p, src2d, dst2d, zeros16)

    h2p = pl.pallas_call(
        _tc_b_body,
        out_shape=jax.ShapeDtypeStruct((NPAD, D_OUT), jnp.float32),
    )(p1, dinv, b1.reshape(1, D_HID), W2)

    p2 = _sc_layer8(h2p, src2d, dst2d, zeros8)

    out = pl.pallas_call(
        _tc_d_body,
        out_shape=jax.ShapeDtypeStruct((N, D_OUT), jnp.float32),
    )(p2, dinv, b2.reshape(1, D_OUT))
    return out


# speedup vs baseline: 1.3512x; 1.3512x over previous
import functools

import jax
import jax.numpy as jnp
from jax import lax
from jax.experimental import pallas as pl
from jax.experimental.pallas import tpu as pltpu
from jax.experimental.pallas import tpu_sc as plsc

N = 10000
E = 320000
D_IN = 128
D_HID = 16
D_OUT = 8

NC = 2
NS = 16
NW = NC * NS
LANE = 128
RPW = 80
EPAD = NW * RPW * LANE
NPAD = 10112
RPT = NPAD // NS
NDEG = NPAD
DPT = NDEG // NS
GRP = 8
NGRP = RPW // GRP

_mesh = plsc.VectorSubcoreMesh(core_axis_name="c", subcore_axis_name="s")
_params = pltpu.CompilerParams(use_tc_tiling_on_sc=False)


@functools.partial(
    pl.kernel,
    out_type=jax.ShapeDtypeStruct((NC * NDEG,), jnp.float32),
    mesh=_mesh,
    compiler_params=_params,
    scratch_types=[
        pltpu.VMEM((RPW, LANE), jnp.int32),
        pltpu.VMEM((LANE,), jnp.float32),
        pltpu.VMEM((DPT,), jnp.float32),
        pltpu.VMEM_SHARED((NDEG,), jnp.float32),
        pltpu.SemaphoreType.DMA,
    ],
)
def _sc_degree(dst_hbm, zeros_hbm, ones_hbm, out_hbm, idx_v, ones_v, buf_v, acc_s, sem):
    c = lax.axis_index("c")
    s = lax.axis_index("s")
    wid = c * NS + s
    pltpu.sync_copy(ones_hbm, ones_v)
    pltpu.sync_copy(dst_hbm.at[pl.ds(wid * RPW, RPW), :], idx_v)
    pltpu.sync_copy(zeros_hbm.at[pl.ds(s * DPT, DPT)], buf_v)
    pltpu.sync_copy(buf_v, acc_s.at[pl.ds(s * DPT, DPT)])
    plsc.subcore_barrier()

    @pl.loop(0, NGRP)
    def _(g):
        for b in range(GRP):
            pltpu.async_copy(ones_v, acc_s.at[idx_v.at[g * GRP + b]], sem, add=True)

    @pl.loop(0, NGRP)
    def _(g):
        for b in range(GRP):
            pltpu.make_async_copy(ones_v, acc_s.at[idx_v.at[g * GRP + b]], sem).wait()

    plsc.subcore_barrier()
    pltpu.sync_copy(acc_s.at[pl.ds(s * DPT, DPT)], buf_v)
    pltpu.sync_copy(buf_v, out_hbm.at[pl.ds(c * NDEG + s * DPT, DPT)])


def _make_sc_layer(F):
    @functools.partial(
        pl.kernel,
        out_type=jax.ShapeDtypeStruct((NC, NPAD, F), jnp.float32),
        mesh=_mesh,
        compiler_params=_params,
        scratch_types=[
            pltpu.VMEM((RPW, LANE), jnp.int32),
            pltpu.VMEM((RPW, LANE), jnp.int32),
            pltpu.VMEM((2, GRP, LANE, F), jnp.float32),
            pltpu.VMEM((RPT, F), jnp.float32),
            pltpu.VMEM_SHARED((NPAD, F), jnp.float32),
            pltpu.VMEM_SHARED((NPAD, F), jnp.float32),
            pltpu.SemaphoreType.DMA,
            pltpu.SemaphoreType.DMA,
        ],
    )
    def _sc_layer(tbl_hbm, src_hbm, dst_hbm, zeros_hbm, out_hbm,
                  src_v, dst_v, msg_v, buf_v, tbl_s, acc_s, gsem, ssem):
        c = lax.axis_index("c")
        s = lax.axis_index("s")
        wid = c * NS + s
        rows = pl.ds(s * RPT, RPT)
        pltpu.sync_copy(src_hbm.at[pl.ds(wid * RPW, RPW), :], src_v)
        pltpu.sync_copy(dst_hbm.at[pl.ds(wid * RPW, RPW), :], dst_v)

        pltpu.sync_copy(tbl_hbm.at[rows, :], buf_v)
        pltpu.sync_copy(buf_v, tbl_s.at[rows, :])

        @pl.when(c == 1)
        def _():
            pltpu.sync_copy(zeros_hbm.at[rows, :], buf_v)

        pltpu.sync_copy(buf_v, acc_s.at[rows, :])
        plsc.subcore_barrier()

        def fire_gathers(g, slot):
            for b in range(GRP):
                pltpu.async_copy(
                    tbl_s.at[src_v.at[g * GRP + b]], msg_v.at[slot, b], gsem)

        fire_gathers(0, 0)

        @pl.loop(0, NGRP)
        def _(g):
            cur = lax.rem(g, 2)
            nxt = lax.rem(g + 1, 2)

            @pl.when(g + 1 < NGRP)
            def _():
                fire_gathers(g + 1, nxt)

            for b in range(GRP):
                pltpu.make_async_copy(
                    tbl_s.at[src_v.at[g * GRP + b]], msg_v.at[cur, b], gsem).wait()
                pltpu.async_copy(
                    msg_v.at[cur, b], acc_s.at[dst_v.at[g * GRP + b]], ssem, add=True)
            for b in range(GRP):
                pltpu.make_async_copy(
                    msg_v.at[cur, b], acc_s.at[dst_v.at[g * GRP + b]], ssem).wait()

        plsc.subcore_barrier()
        pltpu.sync_copy(acc_s.at[rows, :], buf_v)
        pltpu.sync_copy(buf_v, out_hbm.at[c, rows, :])

    return _sc_layer


_sc_layer16 = _make_sc_layer(D_HID)
_sc_layer8 = _make_sc_layer(D_OUT)


def _tc_h1_body(x_ref, w1_ref, h1_ref):
    h = jnp.dot(x_ref[...], w1_ref[...], preferred_element_type=jnp.float32)
    h1_ref[:N, :] = h
    h1_ref[N:, :] = jnp.zeros((NPAD - N, D_HID), jnp.float32)


def _tc_scale_body(deg_ref, h1_ref, dinv_ref, h1p_ref):
    deg = deg_ref[0] + deg_ref[1] + 1.0
    dinv = lax.rsqrt(deg)
    dinv_ref[...] = dinv
    h1p_ref[...] = h1_ref[...] * dinv


def _tc_b_body(p_ref, dinv_ref, b1_ref, w2_ref, h2p_ref):
    sacc = p_ref[0] + p_ref[1]
    dinv = dinv_ref[...]
    z = jnp.maximum(sacc * dinv + b1_ref[...], 0.0)
    h2p_ref[...] = jnp.dot(z, w2_ref[...], preferred_element_type=jnp.float32) * dinv


def _tc_d_body(p_ref, dinv_ref, b2_ref, out_ref):
    sacc = p_ref[0, :N, :] + p_ref[1, :N, :]
    out_ref[...] = sacc * dinv_ref[:N] + b2_ref[...]


def kernel(x, edge_index, W1, b1, W2, b2):
    pad = EPAD - E
    src2d = jnp.concatenate(
        [edge_index[0], jnp.zeros((pad,), jnp.int32)]).reshape(NW * RPW, LANE)
    dst2d = jnp.concatenate(
        [edge_index[1], jnp.full((pad,), N, jnp.int32)]).reshape(NW * RPW, LANE)
    zeros1d = jnp.zeros((NDEG,), jnp.float32)
    ones = jnp.ones((LANE,), jnp.float32)
    zeros16 = jnp.zeros((NPAD, D_HID), jnp.float32)
    zeros8 = jnp.zeros((NPAD, D_OUT), jnp.float32)

    deg_part = _sc_degree(dst2d, zeros1d, ones)
    h1 = pl.pallas_call(
        _tc_h1_body,
        out_shape=jax.ShapeDtypeStruct((NPAD, D_HID), jnp.float32),
    )(x, W1)
    degp = deg_part.reshape(NC, NPAD, 1)

    dinv, h1p = pl.pallas_call(
        _tc_scale_body,
        out_shape=(
            jax.ShapeDtypeStruct((NPAD, 1), jnp.float32),
            jax.ShapeDtypeStruct((NPAD, D_HID), jnp.float32),
        ),
    )(degp, h1)

    p1 = _sc_layer16(h1p, src2d, dst2d, zeros16)

    h2p = pl.pallas_call(
        _tc_b_body,
        out_shape=jax.ShapeDtypeStruct((NPAD, D_OUT), jnp.float32),
    )(p1, dinv, b1.reshape(1, D_HID), W2)

    p2 = _sc_layer8(h2p, src2d, dst2d, zeros8)

    out = pl.pallas_call(
        _tc_d_body,
        out_shape=jax.ShapeDtypeStruct((N, D_OUT), jnp.float32),
    )(p2, dinv, b2.reshape(1, D_OUT))
    return out


# speedup vs baseline: 1.6164x; 1.1963x over previous
import functools

import jax
import jax.numpy as jnp
from jax import lax
from jax.experimental import pallas as pl
from jax.experimental.pallas import tpu as pltpu
from jax.experimental.pallas import tpu_sc as plsc

N = 10000
E = 320000
D_IN = 128
D_HID = 16
D_OUT = 8

NC = 2
NS = 16
NW = NC * NS
LANE = 128
EPW = E // NW
NB = EPW // LANE
TAIL = EPW - NB * LANE
NPAD = 10112
RPT = NPAD // NS
NDEG = NPAD
DPT = NDEG // NS
GRP = 6
NGRP = NB // GRP

_mesh = plsc.VectorSubcoreMesh(core_axis_name="c", subcore_axis_name="s")
_params = pltpu.CompilerParams(use_tc_tiling_on_sc=False)


@functools.partial(
    pl.kernel,
    out_type=jax.ShapeDtypeStruct((NC * NDEG,), jnp.float32),
    mesh=_mesh,
    compiler_params=_params,
    scratch_types=[
        pltpu.VMEM((EPW,), jnp.int32),
        pltpu.VMEM((LANE,), jnp.float32),
        pltpu.VMEM((DPT,), jnp.float32),
        pltpu.VMEM_SHARED((NDEG,), jnp.float32),
        pltpu.SemaphoreType.DMA,
    ],
)
def _sc_degree(edge_hbm, zeros_hbm, ones_hbm, out_hbm, idx_v, ones_v, buf_v, acc_s, sem):
    c = lax.axis_index("c")
    s = lax.axis_index("s")
    wid = c * NS + s
    pltpu.sync_copy(ones_hbm, ones_v)
    pltpu.sync_copy(edge_hbm.at[1, pl.ds(wid * EPW, EPW)], idx_v)
    pltpu.sync_copy(zeros_hbm.at[pl.ds(s * DPT, DPT)], buf_v)
    pltpu.sync_copy(buf_v, acc_s.at[pl.ds(s * DPT, DPT)])
    plsc.subcore_barrier()

    @pl.loop(0, NGRP)
    def _(g):
        for b in range(GRP):
            j = g * GRP + b
            pltpu.async_copy(
                ones_v, acc_s.at[idx_v.at[pl.ds(j * LANE, LANE)]], sem, add=True)

    pltpu.async_copy(
        ones_v.at[pl.ds(0, TAIL)],
        acc_s.at[idx_v.at[pl.ds(NB * LANE, TAIL)]], sem, add=True)

    @pl.loop(0, NGRP)
    def _(g):
        for b in range(GRP):
            j = g * GRP + b
            pltpu.make_async_copy(
                ones_v, acc_s.at[idx_v.at[pl.ds(j * LANE, LANE)]], sem).wait()

    pltpu.make_async_copy(
        ones_v.at[pl.ds(0, TAIL)],
        acc_s.at[idx_v.at[pl.ds(NB * LANE, TAIL)]], sem).wait()

    plsc.subcore_barrier()
    pltpu.sync_copy(acc_s.at[pl.ds(s * DPT, DPT)], buf_v)
    pltpu.sync_copy(buf_v, out_hbm.at[pl.ds(c * NDEG + s * DPT, DPT)])


def _make_sc_layer(F):
    @functools.partial(
        pl.kernel,
        out_type=jax.ShapeDtypeStruct((NC, NPAD, F), jnp.float32),
        mesh=_mesh,
        compiler_params=_params,
        scratch_types=[
            pltpu.VMEM((EPW,), jnp.int32),
            pltpu.VMEM((EPW,), jnp.int32),
            pltpu.VMEM((2, GRP, LANE, F), jnp.float32),
            pltpu.VMEM((RPT, F), jnp.float32),
            pltpu.VMEM_SHARED((NPAD, F), jnp.float32),
            pltpu.VMEM_SHARED((NPAD, F), jnp.float32),
            pltpu.SemaphoreType.DMA,
            pltpu.SemaphoreType.DMA,
        ],
    )
    def _sc_layer(tbl_hbm, edge_hbm, zeros_hbm, out_hbm,
                  src_v, dst_v, msg_v, buf_v, tbl_s, acc_s, gsem, ssem):
        c = lax.axis_index("c")
        s = lax.axis_index("s")
        wid = c * NS + s
        rows = pl.ds(s * RPT, RPT)
        pltpu.sync_copy(edge_hbm.at[0, pl.ds(wid * EPW, EPW)], src_v)
        pltpu.sync_copy(edge_hbm.at[1, pl.ds(wid * EPW, EPW)], dst_v)

        pltpu.sync_copy(tbl_hbm.at[rows, :], buf_v)
        pltpu.sync_copy(buf_v, tbl_s.at[rows, :])

        @pl.when(c == 1)
        def _():
            pltpu.sync_copy(zeros_hbm.at[rows, :], buf_v)

        pltpu.sync_copy(buf_v, acc_s.at[rows, :])
        plsc.subcore_barrier()

        def src_at(j):
            return src_v.at[pl.ds(j * LANE, LANE)]

        def dst_at(j):
            return dst_v.at[pl.ds(j * LANE, LANE)]

        def fire_gathers(g, slot):
            for b in range(GRP):
                pltpu.async_copy(
                    tbl_s.at[src_at(g * GRP + b)], msg_v.at[slot, b], gsem)

        fire_gathers(0, 0)

        @pl.loop(0, NGRP)
        def _(g):
            cur = lax.rem(g, 2)
            nxt = lax.rem(g + 1, 2)

            @pl.when(g + 1 < NGRP)
            def _():
                fire_gathers(g + 1, nxt)

            for b in range(GRP):
                pltpu.make_async_copy(
                    tbl_s.at[src_at(g * GRP + b)], msg_v.at[cur, b], gsem).wait()
                pltpu.async_copy(
                    msg_v.at[cur, b], acc_s.at[dst_at(g * GRP + b)], ssem, add=True)
            for b in range(GRP):
                pltpu.make_async_copy(
                    msg_v.at[cur, b], acc_s.at[dst_at(g * GRP + b)], ssem).wait()

        tsrc = src_v.at[pl.ds(NB * LANE, TAIL)]
        tdst = dst_v.at[pl.ds(NB * LANE, TAIL)]
        tmsg = msg_v.at[0, 0, pl.ds(0, TAIL), :]
        pltpu.sync_copy(tbl_s.at[tsrc], tmsg)
        pltpu.sync_copy(tmsg, acc_s.at[tdst], add=True)

        plsc.subcore_barrier()
        pltpu.sync_copy(acc_s.at[rows, :], buf_v)
        pltpu.sync_copy(buf_v, out_hbm.at[c, rows, :])

    return _sc_layer


_sc_layer16 = _make_sc_layer(D_HID)
_sc_layer8 = _make_sc_layer(D_OUT)


def _tc_h1_body(x_ref, w1_ref, h1_ref):
    h = jnp.dot(x_ref[...], w1_ref[...], preferred_element_type=jnp.float32)
    h1_ref[:N, :] = h
    h1_ref[N:, :] = jnp.zeros((NPAD - N, D_HID), jnp.float32)


def _tc_scale_body(deg_ref, h1_ref, dinv_ref, h1p_ref):
    deg = deg_ref[0] + deg_ref[1] + 1.0
    dinv = lax.rsqrt(deg)
    dinv_ref[...] = dinv
    h1p_ref[...] = h1_ref[...] * dinv


def _tc_b_body(p_ref, dinv_ref, b1_ref, w2_ref, h2p_ref):
    sacc = p_ref[0] + p_ref[1]
    dinv = dinv_ref[...]
    z = jnp.maximum(sacc * dinv + b1_ref[...], 0.0)
    h2p_ref[...] = jnp.dot(z, w2_ref[...], preferred_element_type=jnp.float32) * dinv


def _tc_d_body(p_ref, dinv_ref, b2_ref, out_ref):
    sacc = p_ref[0, :N, :] + p_ref[1, :N, :]
    out_ref[...] = sacc * dinv_ref[:N] + b2_ref[...]


def kernel(x, edge_index, W1, b1, W2, b2):
    zeros1d = jnp.zeros((NDEG,), jnp.float32)
    ones = jnp.ones((LANE,), jnp.float32)
    zeros16 = jnp.zeros((NPAD, D_HID), jnp.float32)
    zeros8 = jnp.zeros((NPAD, D_OUT), jnp.float32)

    deg_part = _sc_degree(edge_index, zeros1d, ones)
    h1 = pl.pallas_call(
        _tc_h1_body,
        out_shape=jax.ShapeDtypeStruct((NPAD, D_HID), jnp.float32),
    )(x, W1)
    degp = deg_part.reshape(NC, NPAD, 1)

    dinv, h1p = pl.pallas_call(
        _tc_scale_body,
        out_shape=(
            jax.ShapeDtypeStruct((NPAD, 1), jnp.float32),
            jax.ShapeDtypeStruct((NPAD, D_HID), jnp.float32),
        ),
    )(degp, h1)

    p1 = _sc_layer16(h1p, edge_index, zeros16)

    h2p = pl.pallas_call(
        _tc_b_body,
        out_shape=jax.ShapeDtypeStruct((NPAD, D_OUT), jnp.float32),
    )(p1, dinv, b1.reshape(1, D_HID), W2)

    p2 = _sc_layer8(h2p, edge_index, zeros8)

    out = pl.pallas_call(
        _tc_d_body,
        out_shape=jax.ShapeDtypeStruct((N, D_OUT), jnp.float32),
    )(p2, dinv, b2.reshape(1, D_OUT))
    return out


# speedup vs baseline: 2.0194x; 1.2493x over previous
import functools

import jax
import jax.numpy as jnp
from jax import lax
from jax.experimental import pallas as pl
from jax.experimental.pallas import tpu as pltpu
from jax.experimental.pallas import tpu_sc as plsc

N = 10000
E = 320000
D_IN = 128
D_HID = 16
D_OUT = 8

NC = 2
NS = 16
NW = NC * NS
LANE = 128
EPW = E // NW
NB = EPW // LANE
TAIL = EPW - NB * LANE
NPAD = 10112
RPT = NPAD // NS
GRP = 6
NGRP = NB // GRP
PK16 = NPAD * D_HID // LANE
PK8 = NPAD * D_OUT // LANE

_mesh = plsc.VectorSubcoreMesh(core_axis_name="c", subcore_axis_name="s")
_params = pltpu.CompilerParams(use_tc_tiling_on_sc=False)


@functools.partial(
    pl.kernel,
    out_type=jax.ShapeDtypeStruct((NC * NPAD, D_HID), jnp.float32),
    mesh=_mesh,
    compiler_params=_params,
    scratch_types=[
        pltpu.VMEM((EPW,), jnp.int32),
        pltpu.VMEM((LANE, D_HID), jnp.float32),
        pltpu.VMEM((RPT, D_HID), jnp.float32),
        pltpu.VMEM_SHARED((NPAD, D_HID), jnp.float32),
        pltpu.SemaphoreType.DMA,
    ],
)
def _sc_degree(edge_hbm, zeros_hbm, ones_hbm, out_hbm, idx_v, ones_v, buf_v, acc_s, sem):
    c = lax.axis_index("c")
    s = lax.axis_index("s")
    wid = c * NS + s
    rows = pl.ds(s * RPT, RPT)
    pltpu.sync_copy(ones_hbm, ones_v)
    pltpu.sync_copy(edge_hbm.at[1, pl.ds(wid * EPW, EPW)], idx_v)
    pltpu.sync_copy(zeros_hbm.at[rows, :], buf_v)
    pltpu.sync_copy(buf_v, acc_s.at[rows, :])
    plsc.subcore_barrier()

    @pl.loop(0, NGRP)
    def _(g):
        for b in range(GRP):
            j = g * GRP + b
            pltpu.async_copy(
                ones_v, acc_s.at[idx_v.at[pl.ds(j * LANE, LANE)]], sem, add=True)

    pltpu.async_copy(
        ones_v.at[pl.ds(0, TAIL), :],
        acc_s.at[idx_v.at[pl.ds(NB * LANE, TAIL)]], sem, add=True)

    @pl.loop(0, NGRP)
    def _(g):
        for b in range(GRP):
            j = g * GRP + b
            pltpu.make_async_copy(
                ones_v, acc_s.at[idx_v.at[pl.ds(j * LANE, LANE)]], sem).wait()

    pltpu.make_async_copy(
        ones_v.at[pl.ds(0, TAIL), :],
        acc_s.at[idx_v.at[pl.ds(NB * LANE, TAIL)]], sem).wait()

    plsc.subcore_barrier()
    pltpu.sync_copy(acc_s.at[rows, :], buf_v)
    pltpu.sync_copy(buf_v, out_hbm.at[pl.ds(c * NPAD + s * RPT, RPT), :])


def _make_sc_layer(F):
    @functools.partial(
        pl.kernel,
        out_type=jax.ShapeDtypeStruct((NC * NPAD, F), jnp.float32),
        mesh=_mesh,
        compiler_params=_params,
        scratch_types=[
            pltpu.VMEM((EPW,), jnp.int32),
            pltpu.VMEM((EPW,), jnp.int32),
            pltpu.VMEM((2, GRP, LANE, F), jnp.float32),
            pltpu.VMEM((RPT, F), jnp.float32),
            pltpu.VMEM_SHARED((NPAD, F), jnp.float32),
            pltpu.VMEM_SHARED((NPAD, F), jnp.float32),
            pltpu.SemaphoreType.DMA,
            pltpu.SemaphoreType.DMA,
        ],
    )
    def _sc_layer(tbl_hbm, edge_hbm, zeros_hbm, out_hbm,
                  src_v, dst_v, msg_v, buf_v, tbl_s, acc_s, gsem, ssem):
        c = lax.axis_index("c")
        s = lax.axis_index("s")
        wid = c * NS + s
        rows = pl.ds(s * RPT, RPT)
        pltpu.sync_copy(edge_hbm.at[0, pl.ds(wid * EPW, EPW)], src_v)
        pltpu.sync_copy(edge_hbm.at[1, pl.ds(wid * EPW, EPW)], dst_v)

        pltpu.sync_copy(tbl_hbm.at[rows, :], buf_v)
        pltpu.sync_copy(buf_v, tbl_s.at[rows, :])

        @pl.when(c == 1)
        def _():
            pltpu.sync_copy(zeros_hbm.at[rows, :], buf_v)

        pltpu.sync_copy(buf_v, acc_s.at[rows, :])
        plsc.subcore_barrier()

        def src_at(j):
            return src_v.at[pl.ds(j * LANE, LANE)]

        def dst_at(j):
            return dst_v.at[pl.ds(j * LANE, LANE)]

        def fire_gathers(g, slot):
            for b in range(GRP):
                pltpu.async_copy(
                    tbl_s.at[src_at(g * GRP + b)], msg_v.at[slot, b], gsem)

        fire_gathers(0, 0)

        @pl.loop(0, NGRP)
        def _(g):
            cur = lax.rem(g, 2)
            nxt = lax.rem(g + 1, 2)

            @pl.when(g + 1 < NGRP)
            def _():
                fire_gathers(g + 1, nxt)

            for b in range(GRP):
                pltpu.make_async_copy(
                    tbl_s.at[src_at(g * GRP + b)], msg_v.at[cur, b], gsem).wait()
                pltpu.async_copy(
                    msg_v.at[cur, b], acc_s.at[dst_at(g * GRP + b)], ssem, add=True)
            for b in range(GRP):
                pltpu.make_async_copy(
                    msg_v.at[cur, b], acc_s.at[dst_at(g * GRP + b)], ssem).wait()

        tsrc = src_v.at[pl.ds(NB * LANE, TAIL)]
        tdst = dst_v.at[pl.ds(NB * LANE, TAIL)]
        tmsg = msg_v.at[0, 0, pl.ds(0, TAIL), :]
        pltpu.sync_copy(tbl_s.at[tsrc], tmsg)
        pltpu.sync_copy(tmsg, acc_s.at[tdst], add=True)

        plsc.subcore_barrier()
        pltpu.sync_copy(acc_s.at[rows, :], buf_v)
        pltpu.sync_copy(buf_v, out_hbm.at[pl.ds(c * NPAD + s * RPT, RPT), :])

    return _sc_layer


_sc_layer16 = _make_sc_layer(D_HID)
_sc_layer8 = _make_sc_layer(D_OUT)


def _pack(a, width):
    g = LANE // width
    a3 = a.reshape(a.shape[0] // g, g, width)
    return jnp.concatenate([a3[:, u, :] for u in range(g)], axis=-1)


def _unpack(a, width):
    g = LANE // width
    a3 = jnp.stack([a[:, u * width:(u + 1) * width] for u in range(g)], axis=1)
    return a3.reshape(a.shape[0] * g, width)


def _tc_h1_body(x_ref, w1_ref, h1pk_ref):
    h = jnp.dot(x_ref[...], w1_ref[...], preferred_element_type=jnp.float32)
    hpad = jnp.concatenate(
        [h, jnp.zeros((NPAD - N, D_HID), jnp.float32)], axis=0)
    h1pk_ref[...] = _pack(hpad, D_HID)


def _tc_scale_body(deg_ref, h1pk_ref, dinvpk_ref, h1ppk_ref):
    deg = deg_ref[0] + deg_ref[1] + 1.0
    dinv = lax.rsqrt(deg)
    dinvpk_ref[...] = dinv
    h1ppk_ref[...] = h1pk_ref[...] * dinv


def _tc_b_body(p_ref, dinvpk_ref, d8pk_ref, b1t_ref, w2_ref, h2ppk_ref):
    sacc = p_ref[0] + p_ref[1]
    z = jnp.maximum(sacc * dinvpk_ref[...] + b1t_ref[...], 0.0)
    z2d = _unpack(z, D_HID)
    h2 = jnp.dot(z2d, w2_ref[...], preferred_element_type=jnp.float32)
    h2ppk_ref[...] = _pack(h2, D_OUT) * d8pk_ref[...]


def _tc_d_body(p_ref, d8pk_ref, b2t_ref, out_ref):
    sacc = p_ref[0] + p_ref[1]
    o = sacc * d8pk_ref[...] + b2t_ref[...]
    out_ref[...] = _unpack(o, D_OUT)[:N, :]


def kernel(x, edge_index, W1, b1, W2, b2):
    zeros16 = jnp.zeros((NPAD, D_HID), jnp.float32)
    zeros8 = jnp.zeros((NPAD, D_OUT), jnp.float32)
    ones2d = jnp.ones((LANE, D_HID), jnp.float32)
    b1t = jnp.tile(b1, 8).reshape(1, LANE)
    b2t = jnp.tile(b2, 16).reshape(1, LANE)

    deg_part = _sc_degree(edge_index, zeros16, ones2d)
    h1pk = pl.pallas_call(
        _tc_h1_body,
        out_shape=jax.ShapeDtypeStruct((PK16, LANE), jnp.float32),
    )(x, W1)
    degv = deg_part.reshape(NC, PK16, LANE)

    dinvpk, h1ppk = pl.pallas_call(
        _tc_scale_body,
        out_shape=(
            jax.ShapeDtypeStruct((PK16, LANE), jnp.float32),
            jax.ShapeDtypeStruct((PK16, LANE), jnp.float32),
        ),
    )(degv, h1pk)

    p1 = _sc_layer16(h1ppk.reshape(NPAD, D_HID), edge_index, zeros16)

    d8pk = dinvpk.reshape(NPAD, D_HID)[:, :D_OUT].reshape(PK8, LANE)

    h2ppk = pl.pallas_call(
        _tc_b_body,
        out_shape=jax.ShapeDtypeStruct((PK8, LANE), jnp.float32),
    )(p1.reshape(NC, PK16, LANE), dinvpk, d8pk, b1t, W2)

    p2 = _sc_layer8(h2ppk.reshape(NPAD, D_OUT), edge_index, zeros8)

    out = pl.pallas_call(
        _tc_d_body,
        out_shape=jax.ShapeDtypeStruct((N, D_OUT), jnp.float32),
    )(p2.reshape(NC, PK8, LANE), d8pk, b2t)
    return out


# speedup vs baseline: 2.5230x; 1.2494x over previous
import functools

import jax
import jax.numpy as jnp
from jax import lax
from jax.experimental import pallas as pl
from jax.experimental.pallas import tpu as pltpu
from jax.experimental.pallas import tpu_sc as plsc

N = 10000
E = 320000
D_IN = 128
D_HID = 16
D_OUT = 8

NC = 2
NS = 16
NW = NC * NS
LANE = 128
EPW = E // NW
NB = EPW // LANE
TAIL = EPW - NB * LANE
NPAD = 10112
RPT = NPAD // NS
GRP = 6
NGRP = NB // GRP
PK16 = NPAD * D_HID // LANE
PK8 = NPAD * D_OUT // LANE

_mesh = plsc.VectorSubcoreMesh(core_axis_name="c", subcore_axis_name="s")
_params = pltpu.CompilerParams(use_tc_tiling_on_sc=False)


@functools.partial(
    pl.kernel,
    out_type=jax.ShapeDtypeStruct((NC * NPAD, D_HID), jnp.float32),
    mesh=_mesh,
    compiler_params=_params,
    scratch_types=[
        pltpu.VMEM((EPW,), jnp.int32),
        pltpu.VMEM((LANE, D_HID), jnp.float32),
        pltpu.VMEM((RPT, D_HID), jnp.float32),
        pltpu.VMEM_SHARED((NPAD, D_HID), jnp.float32),
        pltpu.SemaphoreType.DMA,
    ],
)
def _sc_degree(edge_hbm, zeros_hbm, ones_hbm, out_hbm, idx_v, ones_v, buf_v, acc_s, sem):
    c = lax.axis_index("c")
    s = lax.axis_index("s")
    wid = c * NS + s
    rows = pl.ds(s * RPT, RPT)
    pltpu.sync_copy(ones_hbm, ones_v)
    pltpu.sync_copy(edge_hbm.at[1, pl.ds(wid * EPW, EPW)], idx_v)
    pltpu.sync_copy(zeros_hbm.at[rows, :], buf_v)
    pltpu.sync_copy(buf_v, acc_s.at[rows, :])
    plsc.subcore_barrier()

    @pl.loop(0, NGRP)
    def _(g):
        for b in range(GRP):
            j = g * GRP + b
            pltpu.async_copy(
                ones_v, acc_s.at[idx_v.at[pl.ds(j * LANE, LANE)]], sem, add=True)

    pltpu.async_copy(
        ones_v.at[pl.ds(0, TAIL), :],
        acc_s.at[idx_v.at[pl.ds(NB * LANE, TAIL)]], sem, add=True)

    @pl.loop(0, NGRP)
    def _(g):
        for b in range(GRP):
            j = g * GRP + b
            pltpu.make_async_copy(
                ones_v, acc_s.at[idx_v.at[pl.ds(j * LANE, LANE)]], sem).wait()

    pltpu.make_async_copy(
        ones_v.at[pl.ds(0, TAIL), :],
        acc_s.at[idx_v.at[pl.ds(NB * LANE, TAIL)]], sem).wait()

    plsc.subcore_barrier()
    pltpu.sync_copy(acc_s.at[rows, :], buf_v)
    pltpu.sync_copy(buf_v, out_hbm.at[pl.ds(c * NPAD + s * RPT, RPT), :])


def _make_sc_layer(F):
    @functools.partial(
        pl.kernel,
        out_type=jax.ShapeDtypeStruct((NC * NPAD, F), jnp.float32),
        mesh=_mesh,
        compiler_params=_params,
        scratch_types=[
            pltpu.VMEM((EPW,), jnp.int32),
            pltpu.VMEM((EPW,), jnp.int32),
            pltpu.VMEM((2, GRP, LANE, F), jnp.float32),
            pltpu.VMEM((RPT, F), jnp.float32),
            pltpu.VMEM_SHARED((NPAD, F), jnp.float32),
            pltpu.VMEM_SHARED((NPAD, F), jnp.float32),
            pltpu.SemaphoreType.DMA,
            pltpu.SemaphoreType.DMA,
        ],
    )
    def _sc_layer(tbl_hbm, edge_hbm, zeros_hbm, out_hbm,
                  src_v, dst_v, msg_v, buf_v, tbl_s, acc_s, gsem, ssem):
        c = lax.axis_index("c")
        s = lax.axis_index("s")
        wid = c * NS + s
        rows = pl.ds(s * RPT, RPT)
        pltpu.sync_copy(edge_hbm.at[0, pl.ds(wid * EPW, EPW)], src_v)
        pltpu.sync_copy(edge_hbm.at[1, pl.ds(wid * EPW, EPW)], dst_v)

        pltpu.sync_copy(tbl_hbm.at[rows, :], buf_v)
        pltpu.sync_copy(buf_v, tbl_s.at[rows, :])

        @pl.when(c == 1)
        def _():
            pltpu.sync_copy(zeros_hbm.at[rows, :], buf_v)

        pltpu.sync_copy(buf_v, acc_s.at[rows, :])
        plsc.subcore_barrier()

        def src_at(j):
            return src_v.at[pl.ds(j * LANE, LANE)]

        def dst_at(j):
            return dst_v.at[pl.ds(j * LANE, LANE)]

        def fire_gathers(g, slot):
            for b in range(GRP):
                pltpu.async_copy(
                    tbl_s.at[src_at(g * GRP + b)], msg_v.at[slot, b], gsem)

        fire_gathers(0, 0)

        @pl.loop(0, NGRP)
        def _(g):
            cur = lax.rem(g, 2)
            nxt = lax.rem(g + 1, 2)

            @pl.when(g + 1 < NGRP)
            def _():
                fire_gathers(g + 1, nxt)

            for b in range(GRP):
                pltpu.make_async_copy(
                    tbl_s.at[src_at(g * GRP + b)], msg_v.at[cur, b], gsem).wait()
                pltpu.async_copy(
                    msg_v.at[cur, b], acc_s.at[dst_at(g * GRP + b)], ssem, add=True)
            for b in range(GRP):
                pltpu.make_async_copy(
                    msg_v.at[cur, b], acc_s.at[dst_at(g * GRP + b)], ssem).wait()

        tsrc = src_v.at[pl.ds(NB * LANE, TAIL)]
        tdst = dst_v.at[pl.ds(NB * LANE, TAIL)]
        tmsg = msg_v.at[0, 0, pl.ds(0, TAIL), :]
        pltpu.sync_copy(tbl_s.at[tsrc], tmsg)
        pltpu.sync_copy(tmsg, acc_s.at[tdst], add=True)

        plsc.subcore_barrier()
        pltpu.sync_copy(acc_s.at[rows, :], buf_v)
        pltpu.sync_copy(buf_v, out_hbm.at[pl.ds(c * NPAD + s * RPT, RPT), :])

    return _sc_layer


_sc_layer16 = _make_sc_layer(D_HID)
_sc_layer8 = _make_sc_layer(D_OUT)


def _pack(a, width):
    g = LANE // width
    a3 = a.reshape(a.shape[0] // g, g, width)
    return jnp.concatenate([a3[:, u, :] for u in range(g)], axis=-1)


def _unpack(a, width):
    g = LANE // width
    a3 = jnp.stack([a[:, u * width:(u + 1) * width] for u in range(g)], axis=1)
    return a3.reshape(a.shape[0] * g, width)


def _tc_h1_body(x_ref, w1_ref, h1pk_ref):
    h = jnp.dot(x_ref[...], w1_ref[...], preferred_element_type=jnp.float32)
    hpad = jnp.concatenate(
        [h, jnp.zeros((NPAD - N, D_HID), jnp.float32)], axis=0)
    h1pk_ref[...] = _pack(hpad, D_HID)


def _tc_scale_body(deg_ref, h1pk_ref, dinvpk_ref, h1ppk_ref):
    deg = deg_ref[0] + deg_ref[1] + 1.0
    dinv = lax.rsqrt(deg)
    dinvpk_ref[...] = dinv
    h1ppk_ref[...] = h1pk_ref[...] * dinv


def _tc_b_body(p_ref, dinvpk_ref, d8pk_ref, b1t_ref, w2blk_ref, h2ppk_ref):
    sacc = p_ref[0] + p_ref[1]
    z = jnp.maximum(sacc * dinvpk_ref[...] + b1t_ref[...], 0.0)
    h2r = jnp.dot(z, w2blk_ref[...], preferred_element_type=jnp.float32)
    h23 = h2r.reshape(PK8, 2, 64)
    h2pk = jnp.concatenate([h23[:, 0, :], h23[:, 1, :]], axis=-1)
    h2ppk_ref[...] = h2pk * d8pk_ref[...]


def _tc_d_body(p_ref, d8pk_ref, b2t_ref, out_ref):
    sacc = p_ref[0] + p_ref[1]
    out_ref[...] = sacc * d8pk_ref[...] + b2t_ref[...]


def kernel(x, edge_index, W1, b1, W2, b2):
    zeros16 = jnp.zeros((NPAD, D_HID), jnp.float32)
    zeros8 = jnp.zeros((NPAD, D_OUT), jnp.float32)
    ones2d = jnp.ones((LANE, D_HID), jnp.float32)
    b1t = jnp.tile(b1, 8).reshape(1, LANE)
    b2t = jnp.tile(b2, 16).reshape(1, LANE)
    w2blk = jnp.kron(jnp.eye(8, dtype=jnp.float32), W2)

    deg_part = _sc_degree(edge_index, zeros16, ones2d)
    h1pk = pl.pallas_call(
        _tc_h1_body,
        out_shape=jax.ShapeDtypeStruct((PK16, LANE), jnp.float32),
    )(x, W1)
    degv = deg_part.reshape(NC, PK16, LANE)

    dinvpk, h1ppk = pl.pallas_call(
        _tc_scale_body,
        out_shape=(
            jax.ShapeDtypeStruct((PK16, LANE), jnp.float32),
            jax.ShapeDtypeStruct((PK16, LANE), jnp.float32),
        ),
    )(degv, h1pk)

    p1 = _sc_layer16(h1ppk.reshape(NPAD, D_HID), edge_index, zeros16)

    d8pk = dinvpk.reshape(NPAD, D_HID)[:, :D_OUT].reshape(PK8, LANE)

    h2ppk = pl.pallas_call(
        _tc_b_body,
        out_shape=jax.ShapeDtypeStruct((PK8, LANE), jnp.float32),
    )(p1.reshape(NC, PK16, LANE), dinvpk, d8pk, b1t, w2blk)

    p2 = _sc_layer8(h2ppk.reshape(NPAD, D_OUT), edge_index, zeros8)

    opk = pl.pallas_call(
        _tc_d_body,
        out_shape=jax.ShapeDtypeStruct((PK8, LANE), jnp.float32),
    )(p2.reshape(NC, PK8, LANE), d8pk, b2t)
    return opk.reshape(NPAD, D_OUT)[:N, :]


# speedup vs baseline: 2.5649x; 1.0166x over previous
import functools

import jax
import jax.numpy as jnp
from jax import lax
from jax.experimental import pallas as pl
from jax.experimental.pallas import tpu as pltpu
from jax.experimental.pallas import tpu_sc as plsc

N = 10000
E = 320000
D_IN = 128
D_HID = 16
D_OUT = 8

NC = 2
NS = 16
NW = NC * NS
LANE = 128
EPW = E // NW
NB = EPW // LANE
TAIL = EPW - NB * LANE
NPAD = 10112
RPT = NPAD // NS
GRP = 13
NGRP = NB // GRP
PK16 = NPAD * D_HID // LANE
PK8 = NPAD * D_OUT // LANE

_mesh = plsc.VectorSubcoreMesh(core_axis_name="c", subcore_axis_name="s")
_params = pltpu.CompilerParams(use_tc_tiling_on_sc=False)


@functools.partial(
    pl.kernel,
    out_type=jax.ShapeDtypeStruct((NC * NPAD, D_HID), jnp.float32),
    mesh=_mesh,
    compiler_params=_params,
    scratch_types=[
        pltpu.VMEM((EPW,), jnp.int32),
        pltpu.VMEM((LANE, D_HID), jnp.float32),
        pltpu.VMEM((RPT, D_HID), jnp.float32),
        pltpu.VMEM_SHARED((NPAD, D_HID), jnp.float32),
        pltpu.SemaphoreType.DMA,
    ],
)
def _sc_degree(edge_hbm, zeros_hbm, ones_hbm, out_hbm, idx_v, ones_v, buf_v, acc_s, sem):
    c = lax.axis_index("c")
    s = lax.axis_index("s")
    wid = c * NS + s
    rows = pl.ds(s * RPT, RPT)
    pltpu.sync_copy(ones_hbm, ones_v)
    pltpu.sync_copy(edge_hbm.at[1, pl.ds(wid * EPW, EPW)], idx_v)
    pltpu.sync_copy(zeros_hbm.at[rows, :], buf_v)
    pltpu.sync_copy(buf_v, acc_s.at[rows, :])
    plsc.subcore_barrier()

    @pl.loop(0, NGRP)
    def _(g):
        for b in range(GRP):
            j = g * GRP + b
            pltpu.async_copy(
                ones_v, acc_s.at[idx_v.at[pl.ds(j * LANE, LANE)]], sem, add=True)

    pltpu.async_copy(
        ones_v.at[pl.ds(0, TAIL), :],
        acc_s.at[idx_v.at[pl.ds(NB * LANE, TAIL)]], sem, add=True)

    @pl.loop(0, NGRP)
    def _(g):
        for b in range(GRP):
            j = g * GRP + b
            pltpu.make_async_copy(
                ones_v, acc_s.at[idx_v.at[pl.ds(j * LANE, LANE)]], sem).wait()

    pltpu.make_async_copy(
        ones_v.at[pl.ds(0, TAIL), :],
        acc_s.at[idx_v.at[pl.ds(NB * LANE, TAIL)]], sem).wait()

    plsc.subcore_barrier()
    pltpu.sync_copy(acc_s.at[rows, :], buf_v)
    pltpu.sync_copy(buf_v, out_hbm.at[pl.ds(c * NPAD + s * RPT, RPT), :])


def _make_sc_layer(F):
    @functools.partial(
        pl.kernel,
        out_type=jax.ShapeDtypeStruct((NC * NPAD, F), jnp.float32),
        mesh=_mesh,
        compiler_params=_params,
        scratch_types=[
            pltpu.VMEM((EPW,), jnp.int32),
            pltpu.VMEM((EPW,), jnp.int32),
            pltpu.VMEM((2, GRP, LANE, F), jnp.float32),
            pltpu.VMEM((RPT, F), jnp.float32),
            pltpu.VMEM_SHARED((NPAD, F), jnp.float32),
            pltpu.VMEM_SHARED((NPAD, F), jnp.float32),
            pltpu.SemaphoreType.DMA,
            pltpu.SemaphoreType.DMA,
        ],
    )
    def _sc_layer(tbl_hbm, edge_hbm, zeros_hbm, out_hbm,
                  src_v, dst_v, msg_v, buf_v, tbl_s, acc_s, gsem, ssem):
        c = lax.axis_index("c")
        s = lax.axis_index("s")
        wid = c * NS + s
        rows = pl.ds(s * RPT, RPT)
        pltpu.sync_copy(edge_hbm.at[0, pl.ds(wid * EPW, EPW)], src_v)
        pltpu.sync_copy(edge_hbm.at[1, pl.ds(wid * EPW, EPW)], dst_v)

        pltpu.sync_copy(tbl_hbm.at[rows, :], buf_v)
        pltpu.sync_copy(buf_v, tbl_s.at[rows, :])

        @pl.when(c == 1)
        def _():
            pltpu.sync_copy(zeros_hbm.at[rows, :], buf_v)

        pltpu.sync_copy(buf_v, acc_s.at[rows, :])
        plsc.subcore_barrier()

        def src_at(j):
            return src_v.at[pl.ds(j * LANE, LANE)]

        def dst_at(j):
            return dst_v.at[pl.ds(j * LANE, LANE)]

        def fire_gathers(g, slot):
            for b in range(GRP):
                pltpu.async_copy(
                    tbl_s.at[src_at(g * GRP + b)], msg_v.at[slot, b], gsem)

        fire_gathers(0, 0)

        @pl.loop(0, NGRP)
        def _(g):
            cur = lax.rem(g, 2)
            nxt = lax.rem(g + 1, 2)

            @pl.when(g + 1 < NGRP)
            def _():
                fire_gathers(g + 1, nxt)

            for b in range(GRP):
                pltpu.make_async_copy(
                    tbl_s.at[src_at(g * GRP + b)], msg_v.at[cur, b], gsem).wait()
                pltpu.async_copy(
                    msg_v.at[cur, b], acc_s.at[dst_at(g * GRP + b)], ssem, add=True)
            for b in range(GRP):
                pltpu.make_async_copy(
                    msg_v.at[cur, b], acc_s.at[dst_at(g * GRP + b)], ssem).wait()

        tsrc = src_v.at[pl.ds(NB * LANE, TAIL)]
        tdst = dst_v.at[pl.ds(NB * LANE, TAIL)]
        tmsg = msg_v.at[0, 0, pl.ds(0, TAIL), :]
        pltpu.sync_copy(tbl_s.at[tsrc], tmsg)
        pltpu.sync_copy(tmsg, acc_s.at[tdst], add=True)

        plsc.subcore_barrier()
        pltpu.sync_copy(acc_s.at[rows, :], buf_v)
        pltpu.sync_copy(buf_v, out_hbm.at[pl.ds(c * NPAD + s * RPT, RPT), :])

    return _sc_layer


_sc_layer16 = _make_sc_layer(D_HID)
_sc_layer8 = _make_sc_layer(D_OUT)


def _pack(a, width):
    g = LANE // width
    a3 = a.reshape(a.shape[0] // g, g, width)
    return jnp.concatenate([a3[:, u, :] for u in range(g)], axis=-1)


def _unpack(a, width):
    g = LANE // width
    a3 = jnp.stack([a[:, u * width:(u + 1) * width] for u in range(g)], axis=1)
    return a3.reshape(a.shape[0] * g, width)


def _tc_h1_body(x_ref, w1_ref, h1pk_ref):
    h = jnp.dot(x_ref[...], w1_ref[...], preferred_element_type=jnp.float32)
    hpad = jnp.concatenate(
        [h, jnp.zeros((NPAD - N, D_HID), jnp.float32)], axis=0)
    h1pk_ref[...] = _pack(hpad, D_HID)


def _tc_scale_body(deg_ref, h1pk_ref, dinvpk_ref, h1ppk_ref):
    deg = deg_ref[0] + deg_ref[1] + 1.0
    dinv = lax.rsqrt(deg)
    dinvpk_ref[...] = dinv
    h1ppk_ref[...] = h1pk_ref[...] * dinv


def _tc_b_body(p_ref, dinvpk_ref, d8pk_ref, b1t_ref, w2blk_ref, h2ppk_ref):
    sacc = p_ref[0] + p_ref[1]
    z = jnp.maximum(sacc * dinvpk_ref[...] + b1t_ref[...], 0.0)
    h2r = jnp.dot(z, w2blk_ref[...], preferred_element_type=jnp.float32)
    h23 = h2r.reshape(PK8, 2, 64)
    h2pk = jnp.concatenate([h23[:, 0, :], h23[:, 1, :]], axis=-1)
    h2ppk_ref[...] = h2pk * d8pk_ref[...]


def _tc_d_body(p_ref, d8pk_ref, b2t_ref, out_ref):
    sacc = p_ref[0] + p_ref[1]
    out_ref[...] = sacc * d8pk_ref[...] + b2t_ref[...]


def kernel(x, edge_index, W1, b1, W2, b2):
    zeros16 = jnp.zeros((NPAD, D_HID), jnp.float32)
    zeros8 = jnp.zeros((NPAD, D_OUT), jnp.float32)
    ones2d = jnp.ones((LANE, D_HID), jnp.float32)
    b1t = jnp.tile(b1, 8).reshape(1, LANE)
    b2t = jnp.tile(b2, 16).reshape(1, LANE)
    w2blk = jnp.kron(jnp.eye(8, dtype=jnp.float32), W2)

    deg_part = _sc_degree(edge_index, zeros16, ones2d)
    h1pk = pl.pallas_call(
        _tc_h1_body,
        out_shape=jax.ShapeDtypeStruct((PK16, LANE), jnp.float32),
    )(x, W1)
    degv = deg_part.reshape(NC, PK16, LANE)

    dinvpk, h1ppk = pl.pallas_call(
        _tc_scale_body,
        out_shape=(
            jax.ShapeDtypeStruct((PK16, LANE), jnp.float32),
            jax.ShapeDtypeStruct((PK16, LANE), jnp.float32),
        ),
    )(degv, h1pk)

    p1 = _sc_layer16(h1ppk.reshape(NPAD, D_HID), edge_index, zeros16)

    d8pk = dinvpk.reshape(NPAD, D_HID)[:, :D_OUT].reshape(PK8, LANE)

    h2ppk = pl.pallas_call(
        _tc_b_body,
        out_shape=jax.ShapeDtypeStruct((PK8, LANE), jnp.float32),
    )(p1.reshape(NC, PK16, LANE), dinvpk, d8pk, b1t, w2blk)

    p2 = _sc_layer8(h2ppk.reshape(NPAD, D_OUT), edge_index, zeros8)

    opk = pl.pallas_call(
        _tc_d_body,
        out_shape=jax.ShapeDtypeStruct((PK8, LANE), jnp.float32),
    )(p2.reshape(NC, PK8, LANE), d8pk, b2t)
    return opk.reshape(NPAD, D_OUT)[:N, :]
